# baseline (speedup 1.0000x reference)
import functools

import jax
import jax.numpy as jnp
from jax import lax
from jax.experimental import pallas as pl
from jax.experimental.pallas import tpu as pltpu
from jax.experimental.pallas import tpu_sc as plsc

_V = 1000000
_D = 64
_NC = 2
_NS = 16
_NW = _NC * _NS
_S = 16384
_T = 26
_B = _S * _T
_UPW = _B // (128 * _NW)

_CB = 256
_NFULL = 999936 // _CB
_TAIL0 = _NFULL * _CB

_mesh = plsc.VectorSubcoreMesh(core_axis_name="c", subcore_axis_name="s")
_params = pltpu.CompilerParams(use_tc_tiling_on_sc=True,
                               needs_layout_passes=False)

def _iota16():
    return lax.broadcasted_iota(jnp.int32, (16,), 0)


@functools.partial(
    pl.kernel,
    out_type=jax.ShapeDtypeStruct((_V // 2, 128), jnp.float32),
    mesh=_mesh,
    compiler_params=_params,
    scratch_types=[
        [pltpu.VMEM((_D, _CB), jnp.float32) for _ in range(2)],
        [pltpu.VMEM((_CB // 2, 128), jnp.float32) for _ in range(2)],
        pltpu.VMEM((16, 16), jnp.int32),
        [pltpu.SemaphoreType.DMA for _ in range(2)],
        [pltpu.SemaphoreType.DMA for _ in range(2)],
    ],
)
def _transpose_kernel(wt_hbm, wtail_hbm, wrm_hbm, vin, vout, rot_ref,
                      isem, osem):
    wid = lax.axis_index("s") * _NC + lax.axis_index("c")
    iota = _iota16()
    for t in range(16):
        rot_ref[t, :] = lax.bitwise_and(iota + t, 15)
    base_n = _NFULL // _NW
    extra = _NFULL - base_n * _NW
    nmine = jnp.where(wid < extra, base_n + 1, base_n)
    start = wid * base_n + jnp.minimum(wid, extra)

    def in_slice(b):
        return wt_hbm.at[:, pl.ds((start + b) * _CB, _CB)]

    def out_slice(b):
        return wrm_hbm.at[pl.ds((start + b) * (_CB // 2), _CB // 2), :]

    pvecs = [p0 + iota for p0 in range(0, _CB // 2, 16)]
    cols = [[2 * p0 + iota * 2 + c for c in range(2)]
            for p0 in range(0, _CB // 2, 16)]

    def permute(k):
        def body(th, carry):
            for tk in range(4):
                t = 4 * th + tk
                rot = rot_ref[t, :]
                srows = [j20 + rot for j20 in range(0, 128, 16)]
                rrows = [srows[i] if i < 4 else srows[i] - _D
                         for i in range(8)]
                for p0i in range(len(pvecs)):
                    for j20i in range(8):
                        v = plsc.load_gather(
                            vin[k], [rrows[j20i], cols[p0i][j20i // 4]])
                        plsc.store_scatter(
                            vout[k], [pvecs[p0i], srows[j20i]], v)
            return carry
        lax.fori_loop(0, 4, body, 0)

    for k in range(2):
        @pl.when(k < nmine)
        def _():
            pltpu.async_copy(in_slice(k), vin[k], isem[k])

    def blk(b, k):
        @pl.when(b < nmine)
        def _():
            pltpu.make_async_copy(in_slice(b), vin[k], isem[k]).wait()

            @pl.when(b >= 2)
            def _():
                pltpu.make_async_copy(vout[k], out_slice(b - 2), osem[k]).wait()

            permute(k)
            pltpu.async_copy(vout[k], out_slice(b), osem[k])

            @pl.when(b + 2 < nmine)
            def _():
                pltpu.async_copy(in_slice(b + 2), vin[k], isem[k])

    def pair(i, carry):
        blk(2 * i, 0)
        blk(2 * i + 1, 1)
        return carry

    lax.fori_loop(0, (base_n + 2) // 2, pair, 0)

    for k in range(2):
        @pl.when(k < nmine)
        def _():
            pltpu.make_async_copy(vout[k], out_slice(0), osem[k]).wait()

    @pl.when(wid == _NW - 1)
    def _():
        bounce = vout[0].at[pl.ds(0, 32), :]
        pltpu.async_copy(wtail_hbm, bounce, isem[0])
        pltpu.make_async_copy(wtail_hbm, bounce, isem[0]).wait()
        dst = wrm_hbm.at[pl.ds(_TAIL0 // 2, 32), :]
        pltpu.async_copy(bounce, dst, osem[0])
        pltpu.make_async_copy(bounce, dst, osem[0]).wait()


@functools.partial(
    pl.kernel,
    out_type=jax.ShapeDtypeStruct((_T, _D, _S), jnp.float32),
    mesh=_mesh,
    compiler_params=_params,
    scratch_types=[
        pltpu.VMEM((_UPW, 128), jnp.int32),
        [pltpu.VMEM((128,), jnp.int32) for _ in range(2)],
        [pltpu.VMEM((128,), jnp.int32) for _ in range(2)],
        [pltpu.VMEM((128, 128), jnp.float32) for _ in range(2)],
        [pltpu.VMEM((_D, 128), jnp.float32) for _ in range(2)],
        pltpu.VMEM((16, 16), jnp.int32),
        [pltpu.SemaphoreType.DMA for _ in range(2)],
        [pltpu.SemaphoreType.DMA for _ in range(2)],
        pltpu.SemaphoreType.DMA,
    ],
)
def _gather_kernel(idx_hbm, wrm_hbm, out_hbm, idxv, gv, parv, rows, tbuf,
                   rot_ref, gsem, ssem, stage_sem):
    wid = lax.axis_index("s") * _NC + lax.axis_index("c")
    iota = _iota16()
    for t in range(16):
        rot_ref[t, :] = lax.bitwise_and(iota + t, 15)
    pltpu.async_copy(idx_hbm.at[wid], idxv, stage_sem)
    pltpu.make_async_copy(idx_hbm.at[wid], idxv, stage_sem).wait()

    def compute_idx(b, k):
        for g in range(8):
            v = idxv[b, pl.ds(g * 16, 16)]
            gv[k][pl.ds(g * 16, 16)] = lax.shift_right_logical(v, 1)
            parv[k][pl.ds(g * 16, 16)] = lax.bitwise_and(v, 1) * _D

    def start_gather(k):
        pltpu.async_copy(wrm_hbm.at[gv[k]], rows[k], gsem[k])

    def wait_gather(k):
        pltpu.make_async_copy(wrm_hbm.at[gv[k]], rows[k], gsem[k]).wait()

    def out_slice(b):
        u = wid * _UPW + b
        return out_hbm.at[u // 128, :, pl.ds((u % 128) * 128, 128)]

    def start_store(b, k):
        pltpu.async_copy(tbuf[k], out_slice(b), ssem[k])

    def wait_store(b, k):
        pltpu.make_async_copy(tbuf[k], out_slice(b), ssem[k]).wait()

    jvecs = [j0 + iota for j0 in range(0, _D, 16)]

    def permute(k):
        def body(th, carry):
            for tk in range(4):
                t = 4 * th + tk
                rot = rot_ref[t, :]
                srows = [s0 + rot for s0 in range(0, 128, 16)]
                pars = [plsc.load_gather(parv[k], [srows[i]])
                        for i in range(8)]
                for j0i in range(len(jvecs)):
                    for s0i in range(8):
                        col = pars[s0i] + jvecs[j0i]
                        v = plsc.load_gather(rows[k], [srows[s0i], col])
                        plsc.store_scatter(
                            tbuf[k], [jvecs[j0i], srows[s0i]], v)
            return carry
        lax.fori_loop(0, 4, body, 0)

    for k in range(2):
        compute_idx(k, k)
        start_gather(k)

    def unit(b, k):
        wait_gather(k)

        @pl.when(b >= 2)
        def _():
            wait_store(b - 2, k)

        permute(k)
        start_store(b, k)

        @pl.when(b + 2 < _UPW)
        def _():
            compute_idx(b + 2, k)
            start_gather(k)

    def pair(i, carry):
        unit(2 * i, 0)
        unit(2 * i + 1, 1)
        return carry

    lax.fori_loop(0, _UPW // 2, pair, 0)
    wait_store(_UPW - 2, 0)
    wait_store(_UPW - 1, 1)


def kernel(token_ids, weight):
    wt = weight.T
    wtail = weight[_TAIL0:].reshape(32, 128)
    wrm = _transpose_kernel(wt, wtail)
    idx3 = token_ids.T.reshape(_NW, _UPW, 128).astype(jnp.int32)
    out3 = _gather_kernel(idx3, wrm)
    return jnp.transpose(out3, (2, 0, 1))


# speedup vs baseline: 1.0519x; 1.0519x over previous
import functools

import jax
import jax.numpy as jnp
from jax import lax
from jax.experimental import pallas as pl
from jax.experimental.pallas import tpu as pltpu
from jax.experimental.pallas import tpu_sc as plsc

_V = 1000000
_D = 64
_NC = 2
_NS = 16
_NW = _NC * _NS
_S = 16384
_T = 26
_B = _S * _T
_UPW = _B // (128 * _NW)

_CB = 256
_NFULL = 999936 // _CB
_TAIL0 = _NFULL * _CB

_mesh = plsc.VectorSubcoreMesh(core_axis_name="c", subcore_axis_name="s")
_params = pltpu.CompilerParams(use_tc_tiling_on_sc=True,
                               needs_layout_passes=False)

def _iota16():
    return lax.broadcasted_iota(jnp.int32, (16,), 0)


@functools.partial(
    pl.kernel,
    out_type=jax.ShapeDtypeStruct((_V // 2, 128), jnp.float32),
    mesh=_mesh,
    compiler_params=_params,
    scratch_types=[
        [pltpu.VMEM((_D, _CB), jnp.float32) for _ in range(2)],
        [pltpu.VMEM((_CB // 2, 128), jnp.float32) for _ in range(2)],
        pltpu.VMEM((16, 16), jnp.int32),
        [pltpu.SemaphoreType.DMA for _ in range(2)],
        [pltpu.SemaphoreType.DMA for _ in range(2)],
    ],
)
def _transpose_kernel(wt_hbm, wtail_hbm, wrm_hbm, vin, vout, rot_ref,
                      isem, osem):
    wid = lax.axis_index("s") * _NC + lax.axis_index("c")
    iota = _iota16()
    for t in range(16):
        rot_ref[t, :] = lax.bitwise_and(iota + t, 15)
    base_n = _NFULL // _NW
    extra = _NFULL - base_n * _NW
    nmine = jnp.where(wid < extra, base_n + 1, base_n)
    start = wid * base_n + jnp.minimum(wid, extra)

    def in_slice(b):
        return wt_hbm.at[:, pl.ds((start + b) * _CB, _CB)]

    def out_slice(b):
        return wrm_hbm.at[pl.ds((start + b) * (_CB // 2), _CB // 2), :]

    pvecs = [p0 + iota for p0 in range(0, _CB // 2, 16)]
    cols = [[2 * p0 + iota * 2 + c for c in range(2)]
            for p0 in range(0, _CB // 2, 16)]

    def permute(k):
        def body(th, carry):
            for tk in range(2):
                t = 2 * th + tk
                rot = rot_ref[t, :]
                srows = [j20 + rot for j20 in range(0, 128, 16)]
                rrows = [srows[i] if i < 4 else srows[i] - _D
                         for i in range(8)]
                for p0i in range(len(pvecs)):
                    for j20i in range(8):
                        v = plsc.load_gather(
                            vin[k], [rrows[j20i], cols[p0i][j20i // 4]])
                        plsc.store_scatter(
                            vout[k], [pvecs[p0i], srows[j20i]], v)
            return carry
        lax.fori_loop(0, 8, body, 0)

    for k in range(2):
        @pl.when(k < nmine)
        def _():
            pltpu.async_copy(in_slice(k), vin[k], isem[k])

    def blk(b, k):
        @pl.when(b < nmine)
        def _():
            pltpu.make_async_copy(in_slice(b), vin[k], isem[k]).wait()

            @pl.when(b >= 2)
            def _():
                pltpu.make_async_copy(vout[k], out_slice(b - 2), osem[k]).wait()

            permute(k)
            pltpu.async_copy(vout[k], out_slice(b), osem[k])

            @pl.when(b + 2 < nmine)
            def _():
                pltpu.async_copy(in_slice(b + 2), vin[k], isem[k])

    def pair(i, carry):
        blk(2 * i, 0)
        blk(2 * i + 1, 1)
        return carry

    lax.fori_loop(0, (base_n + 2) // 2, pair, 0)

    for k in range(2):
        @pl.when(k < nmine)
        def _():
            pltpu.make_async_copy(vout[k], out_slice(0), osem[k]).wait()

    @pl.when(wid == _NW - 1)
    def _():
        bounce = vout[0].at[pl.ds(0, 32), :]
        pltpu.async_copy(wtail_hbm, bounce, isem[0])
        pltpu.make_async_copy(wtail_hbm, bounce, isem[0]).wait()
        dst = wrm_hbm.at[pl.ds(_TAIL0 // 2, 32), :]
        pltpu.async_copy(bounce, dst, osem[0])
        pltpu.make_async_copy(bounce, dst, osem[0]).wait()


@functools.partial(
    pl.kernel,
    out_type=jax.ShapeDtypeStruct((_T, _D, _S), jnp.float32),
    mesh=_mesh,
    compiler_params=_params,
    scratch_types=[
        pltpu.VMEM((_UPW, 128), jnp.int32),
        [pltpu.VMEM((128,), jnp.int32) for _ in range(2)],
        [pltpu.VMEM((128,), jnp.int32) for _ in range(2)],
        [pltpu.VMEM((128, 128), jnp.float32) for _ in range(2)],
        [pltpu.VMEM((_D, 128), jnp.float32) for _ in range(2)],
        pltpu.VMEM((16, 16), jnp.int32),
        [pltpu.SemaphoreType.DMA for _ in range(2)],
        [pltpu.SemaphoreType.DMA for _ in range(2)],
        pltpu.SemaphoreType.DMA,
    ],
)
def _gather_kernel(idx_hbm, wrm_hbm, out_hbm, idxv, gv, parv, rows, tbuf,
                   rot_ref, gsem, ssem, stage_sem):
    wid = lax.axis_index("s") * _NC + lax.axis_index("c")
    iota = _iota16()
    for t in range(16):
        rot_ref[t, :] = lax.bitwise_and(iota + t, 15)
    pltpu.async_copy(idx_hbm.at[wid], idxv, stage_sem)
    pltpu.make_async_copy(idx_hbm.at[wid], idxv, stage_sem).wait()

    def compute_idx(b, k):
        for g in range(8):
            v = idxv[b, pl.ds(g * 16, 16)]
            gv[k][pl.ds(g * 16, 16)] = lax.shift_right_logical(v, 1)
            parv[k][pl.ds(g * 16, 16)] = lax.bitwise_and(v, 1) * _D

    def start_gather(k):
        pltpu.async_copy(wrm_hbm.at[gv[k]], rows[k], gsem[k])

    def wait_gather(k):
        pltpu.make_async_copy(wrm_hbm.at[gv[k]], rows[k], gsem[k]).wait()

    def out_slice(b):
        u = wid * _UPW + b
        return out_hbm.at[u // 128, :, pl.ds((u % 128) * 128, 128)]

    def start_store(b, k):
        pltpu.async_copy(tbuf[k], out_slice(b), ssem[k])

    def wait_store(b, k):
        pltpu.make_async_copy(tbuf[k], out_slice(b), ssem[k]).wait()

    jvecs = [j0 + iota for j0 in range(0, _D, 16)]

    def permute(k):
        def body(th, carry):
            for tk in range(2):
                t = 2 * th + tk
                rot = rot_ref[t, :]
                srows = [s0 + rot for s0 in range(0, 128, 16)]
                pars = [plsc.load_gather(parv[k], [srows[i]])
                        for i in range(8)]
                for j0i in range(len(jvecs)):
                    for s0i in range(8):
                        col = pars[s0i] + jvecs[j0i]
                        v = plsc.load_gather(rows[k], [srows[s0i], col])
                        plsc.store_scatter(
                            tbuf[k], [jvecs[j0i], srows[s0i]], v)
            return carry
        lax.fori_loop(0, 8, body, 0)

    for k in range(2):
        compute_idx(k, k)
        start_gather(k)

    def unit(b, k):
        wait_gather(k)

        @pl.when(b >= 2)
        def _():
            wait_store(b - 2, k)

        permute(k)
        start_store(b, k)

        @pl.when(b + 2 < _UPW)
        def _():
            compute_idx(b + 2, k)
            start_gather(k)

    def pair(i, carry):
        unit(2 * i, 0)
        unit(2 * i + 1, 1)
        return carry

    lax.fori_loop(0, _UPW // 2, pair, 0)
    wait_store(_UPW - 2, 0)
    wait_store(_UPW - 1, 1)


def kernel(token_ids, weight):
    wt = weight.T
    wtail = weight[_TAIL0:].reshape(32, 128)
    wrm = _transpose_kernel(wt, wtail)
    idx3 = token_ids.T.reshape(_NW, _UPW, 128).astype(jnp.int32)
    out3 = _gather_kernel(idx3, wrm)
    return jnp.transpose(out3, (2, 0, 1))
